# Initial kernel scaffold; baseline (speedup 1.0000x reference)
import jax
import jax.numpy as jnp
from jax.experimental import pallas as pl


def kernel(x, table_0, table_1, table_2, table_3, table_4, enc_w, enc_b):
    raise NotImplementedError("write your pallas kernel here")


# speedup vs baseline: 4.1088x; 4.1088x over previous
import functools

import jax
import jax.numpy as jnp
from jax import lax
from jax.experimental import pallas as pl
from jax.experimental.pallas import tpu as pltpu
from jax.experimental.pallas import tpu_sc as plsc

_EMB_DIMS = (32, 32, 64, 32, 32)
_OFFS = (0, 32, 64, 128, 160)
_TOTAL = 192
_MODEL = 768
_LG = 128


def _gather_concat(x3, t0, t1, t2, t3, t4):
    n_groups = x3.shape[1]
    n_tok = n_groups * _LG
    info = plsc.get_sparse_core_info()
    nc = info.num_cores
    nw = nc * info.num_subcores
    g_per_w = n_groups // nw
    mesh = plsc.VectorSubcoreMesh(core_axis_name="c", subcore_axis_name="s")

    scratch = (
        [pltpu.VMEM((g_per_w, _LG), jnp.int32) for _ in range(5)]
        + [pltpu.VMEM((_LG, d), jnp.float32) for d in _EMB_DIMS]
        + [pltpu.VMEM((_LG, d), jnp.float32) for d in _EMB_DIMS]
        + [pltpu.SemaphoreType.DMA,
           pltpu.SemaphoreType.DMA,
           pltpu.SemaphoreType.DMA]
    )

    @functools.partial(
        pl.kernel,
        mesh=mesh,
        out_type=jax.ShapeDtypeStruct((n_tok, _TOTAL), jnp.float32),
        scratch_types=scratch,
        compiler_params=pltpu.CompilerParams(use_tc_tiling_on_sc=False),
    )
    def k(x_ref, r0, r1, r2, r3, r4, h_ref,
          i0, i1, i2, i3, i4,
          a0, a1, a2, a3, a4,
          b0, b1, b2, b3, b4,
          sg, ss0, ss1):
        tbls = (r0, r1, r2, r3, r4)
        idxs = (i0, i1, i2, i3, i4)
        rows = ((a0, a1, a2, a3, a4), (b0, b1, b2, b3, b4))
        ssems = (ss0, ss1)
        wid = lax.axis_index("s") * nc + lax.axis_index("c")
        g0 = wid * g_per_w
        for f in range(5):
            pltpu.sync_copy(x_ref.at[f, pl.ds(g0, g_per_w)], idxs[f])
        pending = [None, None]
        for it in range(g_per_w):
            s = it % 2
            if pending[s] is not None:
                for cp in pending[s]:
                    cp.wait()
            gathers = [
                pltpu.async_copy(tbls[f].at[idxs[f].at[it]], rows[s][f], sg)
                for f in range(5)
            ]
            for cp in gathers:
                cp.wait()
            row0 = (g0 + it) * _LG
            pending[s] = [
                pltpu.async_copy(
                    rows[s][f],
                    h_ref.at[pl.ds(row0, _LG), pl.ds(_OFFS[f], _EMB_DIMS[f])],
                    ssems[s])
                for f in range(5)
            ]
        for s in range(2):
            if pending[s] is not None:
                for cp in pending[s]:
                    cp.wait()

    return k(x3, t0, t1, t2, t3, t4)


def _encode(h, enc_w, enc_b2):
    n_tok = h.shape[0]
    blk = 1024

    def body(h_ref, w_ref, b_ref, o_ref):
        o_ref[...] = (
            jnp.dot(h_ref[...], w_ref[...], preferred_element_type=jnp.float32)
            + b_ref[...]
        )

    return pl.pallas_call(
        body,
        grid=(n_tok // blk,),
        in_specs=[
            pl.BlockSpec((blk, _TOTAL), lambda i: (i, 0)),
            pl.BlockSpec((_TOTAL, _MODEL), lambda i: (0, 0)),
            pl.BlockSpec((1, _MODEL), lambda i: (0, 0)),
        ],
        out_specs=pl.BlockSpec((blk, _MODEL), lambda i: (i, 0)),
        out_shape=jax.ShapeDtypeStruct((n_tok, _MODEL), jnp.float32),
    )(h, enc_w, enc_b2)


def kernel(x, table_0, table_1, table_2, table_3, table_4, enc_w, enc_b):
    b, s, f = x.shape
    n_tok = b * s
    x3 = x.astype(jnp.int32).reshape(n_tok // _LG, _LG, f).transpose(2, 0, 1)
    h = _gather_concat(x3, table_0, table_1, table_2, table_3, table_4)
    out = _encode(h, enc_w, enc_b.reshape(1, _MODEL))
    return out.reshape(b, s, _MODEL)


# speedup vs baseline: 4.3321x; 1.0543x over previous
import functools

import jax
import jax.numpy as jnp
from jax import lax
from jax.experimental import pallas as pl
from jax.experimental.pallas import tpu as pltpu
from jax.experimental.pallas import tpu_sc as plsc

_EMB_DIMS = (32, 32, 64, 32, 32)
_OFFS = (0, 32, 64, 128, 160)
_TOTAL = 192
_MODEL = 768
_LG = 128


def _gather_concat(x3, t0, t1, t2, t3, t4):
    n_groups = x3.shape[1]
    n_tok = n_groups * _LG
    info = plsc.get_sparse_core_info()
    nc = info.num_cores
    nw = nc * info.num_subcores
    g_per_w = n_groups // nw
    mesh = plsc.VectorSubcoreMesh(core_axis_name="c", subcore_axis_name="s")

    scratch = (
        [pltpu.VMEM((g_per_w, _LG), jnp.int32) for _ in range(5)]
        + [pltpu.VMEM((_LG, d), jnp.float32) for d in _EMB_DIMS]
        + [pltpu.VMEM((_LG, d), jnp.float32) for d in _EMB_DIMS]
        + [pltpu.SemaphoreType.DMA,
           pltpu.SemaphoreType.DMA,
           pltpu.SemaphoreType.DMA]
    )

    @functools.partial(
        pl.kernel,
        mesh=mesh,
        out_type=jax.ShapeDtypeStruct((n_tok, _TOTAL), jnp.float32),
        scratch_types=scratch,
        compiler_params=pltpu.CompilerParams(use_tc_tiling_on_sc=False),
    )
    def k(x_ref, r0, r1, r2, r3, r4, h_ref,
          i0, i1, i2, i3, i4,
          a0, a1, a2, a3, a4,
          b0, b1, b2, b3, b4,
          sg, ss0, ss1):
        tbls = (r0, r1, r2, r3, r4)
        idxs = (i0, i1, i2, i3, i4)
        rows = ((a0, a1, a2, a3, a4), (b0, b1, b2, b3, b4))
        ssems = (ss0, ss1)
        wid = lax.axis_index("s") * nc + lax.axis_index("c")
        g0 = wid * g_per_w
        for f in range(5):
            pltpu.sync_copy(x_ref.at[f, pl.ds(g0, g_per_w)], idxs[f])
        pending = [None, None]
        for it in range(g_per_w):
            s = it % 2
            if pending[s] is not None:
                for cp in pending[s]:
                    cp.wait()
            gathers = [
                pltpu.async_copy(tbls[f].at[idxs[f].at[it]], rows[s][f], sg)
                for f in range(5)
            ]
            for cp in gathers:
                cp.wait()
            row0 = (g0 + it) * _LG
            pending[s] = [
                pltpu.async_copy(
                    rows[s][f],
                    h_ref.at[pl.ds(row0, _LG), pl.ds(_OFFS[f], _EMB_DIMS[f])],
                    ssems[s])
                for f in range(5)
            ]
        for s in range(2):
            if pending[s] is not None:
                for cp in pending[s]:
                    cp.wait()

    return k(x3, t0, t1, t2, t3, t4)


def _encode(h, enc_w, enc_b2):
    n_tok = h.shape[0]
    blk = 2048

    def body(h_ref, w_ref, b_ref, o_ref):
        o_ref[...] = (
            jnp.dot(h_ref[...], w_ref[...], preferred_element_type=jnp.float32)
            + b_ref[...]
        )

    return pl.pallas_call(
        body,
        grid=(n_tok // blk,),
        in_specs=[
            pl.BlockSpec((blk, _TOTAL), lambda i: (i, 0)),
            pl.BlockSpec((_TOTAL, _MODEL), lambda i: (0, 0)),
            pl.BlockSpec((1, _MODEL), lambda i: (0, 0)),
        ],
        out_specs=pl.BlockSpec((blk, _MODEL), lambda i: (i, 0)),
        out_shape=jax.ShapeDtypeStruct((n_tok, _MODEL), jnp.float32),
    )(h, enc_w, enc_b2)


def kernel(x, table_0, table_1, table_2, table_3, table_4, enc_w, enc_b):
    b, s, f = x.shape
    n_tok = b * s
    x3 = x.astype(jnp.int32).reshape(n_tok // _LG, _LG, f).transpose(2, 0, 1)
    h = _gather_concat(x3, table_0, table_1, table_2, table_3, table_4)
    out = _encode(h, enc_w, enc_b.reshape(1, _MODEL))
    return out.reshape(b, s, _MODEL)


# speedup vs baseline: 4.3788x; 1.0108x over previous
import functools

import jax
import jax.numpy as jnp
from jax import lax
from jax.experimental import pallas as pl
from jax.experimental.pallas import tpu as pltpu
from jax.experimental.pallas import tpu_sc as plsc

_EMB_DIMS = (32, 32, 64, 32, 32)
_OFFS = (0, 32, 64, 128, 160)
_TOTAL = 192
_MODEL = 768
_LG = 128


def _gather_concat(x3, t0, t1, t2, t3, t4):
    n_groups = x3.shape[1]
    n_tok = n_groups * _LG
    info = plsc.get_sparse_core_info()
    nc = info.num_cores
    nw = nc * info.num_subcores
    g_per_w = n_groups // nw
    mesh = plsc.VectorSubcoreMesh(core_axis_name="c", subcore_axis_name="s")

    scratch = (
        [pltpu.VMEM((g_per_w, _LG), jnp.int32) for _ in range(5)]
        + [pltpu.VMEM((_LG, d), jnp.float32) for d in _EMB_DIMS]
        + [pltpu.VMEM((_LG, d), jnp.float32) for d in _EMB_DIMS]
        + [pltpu.SemaphoreType.DMA,
           pltpu.SemaphoreType.DMA,
           pltpu.SemaphoreType.DMA]
    )

    @functools.partial(
        pl.kernel,
        mesh=mesh,
        out_type=jax.ShapeDtypeStruct((n_tok, _TOTAL), jnp.float32),
        scratch_types=scratch,
        compiler_params=pltpu.CompilerParams(use_tc_tiling_on_sc=False),
    )
    def k(x_ref, r0, r1, r2, r3, r4, h_ref,
          i0, i1, i2, i3, i4,
          a0, a1, a2, a3, a4,
          b0, b1, b2, b3, b4,
          sg, ss0, ss1):
        tbls = (r0, r1, r2, r3, r4)
        idxs = (i0, i1, i2, i3, i4)
        rows = ((a0, a1, a2, a3, a4), (b0, b1, b2, b3, b4))
        ssems = (ss0, ss1)
        wid = lax.axis_index("s") * nc + lax.axis_index("c")
        g0 = wid * g_per_w
        for f in range(5):
            pltpu.sync_copy(x_ref.at[f, pl.ds(g0, g_per_w)], idxs[f])
        pending = [None, None]
        for it in range(g_per_w):
            s = it % 2
            if pending[s] is not None:
                for cp in pending[s]:
                    cp.wait()
            gathers = [
                pltpu.async_copy(tbls[f].at[idxs[f].at[it]], rows[s][f], sg)
                for f in range(5)
            ]
            for cp in gathers:
                cp.wait()
            row0 = (g0 + it) * _LG
            pending[s] = [
                pltpu.async_copy(
                    rows[s][f],
                    h_ref.at[pl.ds(row0, _LG), pl.ds(_OFFS[f], _EMB_DIMS[f])],
                    ssems[s])
                for f in range(5)
            ]
        for s in range(2):
            if pending[s] is not None:
                for cp in pending[s]:
                    cp.wait()

    return k(x3, t0, t1, t2, t3, t4)


def _encode(h, enc_w, enc_b2):
    n_tok = h.shape[0]
    blk = 4096

    def body(h_ref, w_ref, b_ref, o_ref):
        o_ref[...] = (
            jnp.dot(h_ref[...], w_ref[...], preferred_element_type=jnp.float32)
            + b_ref[...]
        )

    return pl.pallas_call(
        body,
        grid=(n_tok // blk,),
        in_specs=[
            pl.BlockSpec((blk, _TOTAL), lambda i: (i, 0)),
            pl.BlockSpec((_TOTAL, _MODEL), lambda i: (0, 0)),
            pl.BlockSpec((1, _MODEL), lambda i: (0, 0)),
        ],
        out_specs=pl.BlockSpec((blk, _MODEL), lambda i: (i, 0)),
        out_shape=jax.ShapeDtypeStruct((n_tok, _MODEL), jnp.float32),
    )(h, enc_w, enc_b2)


def kernel(x, table_0, table_1, table_2, table_3, table_4, enc_w, enc_b):
    b, s, f = x.shape
    n_tok = b * s
    x3 = x.astype(jnp.int32).reshape(n_tok // _LG, _LG, f).transpose(2, 0, 1)
    h = _gather_concat(x3, table_0, table_1, table_2, table_3, table_4)
    out = _encode(h, enc_w, enc_b.reshape(1, _MODEL))
    return out.reshape(b, s, _MODEL)
